# Initial kernel scaffold; baseline (speedup 1.0000x reference)
import jax
import jax.numpy as jnp
from jax.experimental import pallas as pl


def kernel(x, edge_index, W_in, b_in, W_msg0, b_msg0, W_upd0, b_upd0, g0, be0, W_msg1, b_msg1, W_upd1, b_upd1, g1, be1, W_o1, b_o1, W_o2, b_o2):
    raise NotImplementedError("write your pallas kernel here")


# speedup vs baseline: 8.1511x; 8.1511x over previous
import functools

import jax
import jax.numpy as jnp
from jax import lax
from jax.experimental import pallas as pl
from jax.experimental.pallas import tpu as pltpu
from jax.experimental.pallas import tpu_sc as plsc

_N = 10000
_E = 640000
_D = 128
_H = 64
_W = 80
_NP = 10240

_NC = 2
_NS = 16
_NW = _NC * _NS
_CHUNK = 128
_CPW = 160
_EP = _NW * _CPW * _CHUNK
_RPS = _NP // _NS

_BR = 256


def _edge_pass_body(hext, row2d, col2d, zrows, out, idx_r, idx_c, rows, acc,
                    gsem):
    c = lax.axis_index("c")
    s = lax.axis_index("s")
    wid = c * _NS + s

    pltpu.sync_copy(zrows, acc.at[pl.ds(s * _RPS, _RPS)])
    pltpu.sync_copy(row2d.at[pl.ds(wid * _CPW, _CPW)], idx_r)
    pltpu.sync_copy(col2d.at[pl.ds(wid * _CPW, _CPW)], idx_c)
    plsc.subcore_barrier()

    def body(j, carry):
        pltpu.async_copy(hext.at[idx_r.at[j]], rows, gsem).wait()
        pltpu.sync_copy(rows, acc.at[idx_c.at[j]], add=True)
        return carry

    lax.fori_loop(0, _CPW, body, 0)

    plsc.subcore_barrier()
    pltpu.sync_copy(acc.at[pl.ds(s * _RPS, _RPS)],
                    out.at[c, pl.ds(s * _RPS, _RPS)])


@functools.lru_cache(maxsize=None)
def _get_edge_pass():
    mesh = plsc.VectorSubcoreMesh(core_axis_name="c", subcore_axis_name="s",
                                  num_cores=_NC, num_subcores=_NS)
    return pl.kernel(
        _edge_pass_body,
        out_type=jax.ShapeDtypeStruct((_NC, _NP, _W), jnp.float32),
        mesh=mesh,
        scratch_types=[
            pltpu.VMEM((_CPW, _CHUNK), jnp.int32),
            pltpu.VMEM((_CPW, _CHUNK), jnp.int32),
            pltpu.VMEM((_CHUNK, _W), jnp.float32),
            pltpu.VMEM_SHARED((_NP, _W), jnp.float32),
            pltpu.SemaphoreType.DMA,
        ],
        compiler_params=pltpu.CompilerParams(use_tc_tiling_on_sc=False),
    )


def _embed_body(x_ref, w_ref, b_ref, o_ref):
    h = jnp.dot(x_ref[...], w_ref[...],
                preferred_element_type=jnp.float32) + b_ref[...]
    ones = jnp.ones((_BR, 1), jnp.float32)
    zer = jnp.zeros((_BR, _W - _H - 1), jnp.float32)
    o_ref[...] = jnp.concatenate([h, ones, zer], axis=1)


_embed = pl.pallas_call(
    _embed_body,
    grid=(_NP // _BR,),
    in_specs=[
        pl.BlockSpec((_BR, _D), lambda i: (i, 0)),
        pl.BlockSpec((_D, _H), lambda i: (0, 0)),
        pl.BlockSpec((1, _H), lambda i: (0, 0)),
    ],
    out_specs=pl.BlockSpec((_BR, _W), lambda i: (i, 0)),
    out_shape=jax.ShapeDtypeStruct((_NP, _W), jnp.float32),
)


def _new_h(he_ref, p_ref, wma_ref, wmb_ref, bm_ref, wua_ref, wub_ref, bu_ref,
           g_ref, be_ref):
    h = he_ref[...][:, :_H]
    p0 = p_ref[0]
    p1 = p_ref[1]
    G = p0[:, :_H] + p1[:, :_H]
    cnt = p0[:, _H:_H + 1] + p1[:, _H:_H + 1]
    cntc = jnp.maximum(cnt, 1.0)
    aggr = (jnp.dot(G, wma_ref[...], preferred_element_type=jnp.float32)
            + cnt * jnp.dot(h, wmb_ref[...], preferred_element_type=jnp.float32)
            + cnt * bm_ref[...]) / cntc
    upd = (jnp.dot(h, wua_ref[...], preferred_element_type=jnp.float32)
           + jnp.dot(aggr, wub_ref[...], preferred_element_type=jnp.float32)
           + bu_ref[...])
    y = h + upd
    mu = jnp.mean(y, axis=1, keepdims=True)
    var = jnp.mean((y - mu) ** 2, axis=1, keepdims=True)
    return (y - mu) * lax.rsqrt(var + 1e-5) * g_ref[...] + be_ref[...]


def _layer_body(he_ref, p_ref, wma_ref, wmb_ref, bm_ref, wua_ref, wub_ref,
                bu_ref, g_ref, be_ref, o_ref):
    hn = _new_h(he_ref, p_ref, wma_ref, wmb_ref, bm_ref, wua_ref, wub_ref,
                bu_ref, g_ref, be_ref)
    ones = jnp.ones((_BR, 1), jnp.float32)
    zer = jnp.zeros((_BR, _W - _H - 1), jnp.float32)
    o_ref[...] = jnp.concatenate([hn, ones, zer], axis=1)


def _final_body(he_ref, p_ref, wma_ref, wmb_ref, bm_ref, wua_ref, wub_ref,
                bu_ref, g_ref, be_ref, wo1_ref, bo1_ref, wo2_ref, bo2_ref,
                o_ref):
    hn = _new_h(he_ref, p_ref, wma_ref, wmb_ref, bm_ref, wua_ref, wub_ref,
                bu_ref, g_ref, be_ref)
    z = jnp.dot(hn, wo1_ref[...], preferred_element_type=jnp.float32) \
        + bo1_ref[...]
    z = 0.5 * z * (1.0 + lax.erf(z * 0.7071067811865476))
    z = jnp.dot(z, wo2_ref[...], preferred_element_type=jnp.float32) \
        + bo2_ref[...]
    o_ref[...] = 1.0 / (1.0 + jnp.exp(-z))


_layer_specs = [
    pl.BlockSpec((_BR, _W), lambda i: (i, 0)),
    pl.BlockSpec((_NC, _BR, _W), lambda i: (0, i, 0)),
    pl.BlockSpec((_H, _H), lambda i: (0, 0)),
    pl.BlockSpec((_H, _H), lambda i: (0, 0)),
    pl.BlockSpec((1, _H), lambda i: (0, 0)),
    pl.BlockSpec((_H, _H), lambda i: (0, 0)),
    pl.BlockSpec((_H, _H), lambda i: (0, 0)),
    pl.BlockSpec((1, _H), lambda i: (0, 0)),
    pl.BlockSpec((1, _H), lambda i: (0, 0)),
    pl.BlockSpec((1, _H), lambda i: (0, 0)),
]

_layer = pl.pallas_call(
    _layer_body,
    grid=(_NP // _BR,),
    in_specs=_layer_specs,
    out_specs=pl.BlockSpec((_BR, _W), lambda i: (i, 0)),
    out_shape=jax.ShapeDtypeStruct((_NP, _W), jnp.float32),
)

_final = pl.pallas_call(
    _final_body,
    grid=(_NP // _BR,),
    in_specs=_layer_specs + [
        pl.BlockSpec((_H, _H // 2), lambda i: (0, 0)),
        pl.BlockSpec((1, _H // 2), lambda i: (0, 0)),
        pl.BlockSpec((_H // 2, 1), lambda i: (0, 0)),
        pl.BlockSpec((1, 1), lambda i: (0, 0)),
    ],
    out_specs=pl.BlockSpec((_BR, 1), lambda i: (i, 0)),
    out_shape=jax.ShapeDtypeStruct((_NP, 1), jnp.float32),
)


def kernel(x, edge_index, W_in, b_in, W_msg0, b_msg0, W_upd0, b_upd0, g0, be0,
           W_msg1, b_msg1, W_upd1, b_upd1, g1, be1, W_o1, b_o1, W_o2, b_o2):
    row = edge_index[0]
    col = edge_index[1]
    pad = jnp.full((_EP - _E,), _N, jnp.int32)
    row2d = jnp.concatenate([row, pad]).reshape(_EP // _CHUNK, _CHUNK)
    col2d = jnp.concatenate([col, pad]).reshape(_EP // _CHUNK, _CHUNK)
    xp = jnp.zeros((_NP, _D), jnp.float32).at[:_N].set(x)
    zrows = jnp.zeros((_RPS, _W), jnp.float32)

    edge_pass = _get_edge_pass()
    h0e = _embed(xp, W_in, b_in.reshape(1, _H))
    p0 = edge_pass(h0e, row2d, col2d, zrows)
    h1e = _layer(h0e, p0,
                 W_msg0[:_H], W_msg0[_H:], b_msg0.reshape(1, _H),
                 W_upd0[:_H], W_upd0[_H:], b_upd0.reshape(1, _H),
                 g0.reshape(1, _H), be0.reshape(1, _H))
    p1 = edge_pass(h1e, row2d, col2d, zrows)
    out = _final(h1e, p1,
                 W_msg1[:_H], W_msg1[_H:], b_msg1.reshape(1, _H),
                 W_upd1[:_H], W_upd1[_H:], b_upd1.reshape(1, _H),
                 g1.reshape(1, _H), be1.reshape(1, _H),
                 W_o1, b_o1.reshape(1, _H // 2), W_o2, b_o2.reshape(1, 1))
    return out[:_N, 0]


# speedup vs baseline: 8.8207x; 1.0821x over previous
import functools

import jax
import jax.numpy as jnp
from jax import lax
from jax.experimental import pallas as pl
from jax.experimental.pallas import tpu as pltpu
from jax.experimental.pallas import tpu_sc as plsc

_N = 10000
_E = 640000
_D = 128
_H = 64
_W = 80
_NP = 10240

_NC = 2
_NS = 16
_NW = _NC * _NS
_CHUNK = 128
_CPW = 160
_EP = _NW * _CPW * _CHUNK
_RPS = _NP // _NS

_BR = 256


def _edge_pass_body(hext, row2d, col2d, zrows, out, idx_r, idx_c, rows0, rows1,
                    acc, gsem0, gsem1):
    c = lax.axis_index("c")
    s = lax.axis_index("s")
    wid = c * _NS + s

    pltpu.sync_copy(zrows, acc.at[pl.ds(s * _RPS, _RPS)])
    pltpu.sync_copy(row2d.at[pl.ds(wid * _CPW, _CPW)], idx_r)
    pltpu.sync_copy(col2d.at[pl.ds(wid * _CPW, _CPW)], idx_c)
    plsc.subcore_barrier()

    nh = _CPW // 2
    pltpu.async_copy(hext.at[idx_r.at[0]], rows0, gsem0)

    def body(i, carry):
        j = 2 * i
        pltpu.make_async_copy(hext.at[idx_r.at[j]], rows0, gsem0).wait()
        pltpu.async_copy(hext.at[idx_r.at[j + 1]], rows1, gsem1)
        pltpu.sync_copy(rows0, acc.at[idx_c.at[j]], add=True)
        pltpu.make_async_copy(hext.at[idx_r.at[j + 1]], rows1, gsem1).wait()

        @pl.when(i < nh - 1)
        def _():
            pltpu.async_copy(hext.at[idx_r.at[j + 2]], rows0, gsem0)

        pltpu.sync_copy(rows1, acc.at[idx_c.at[j + 1]], add=True)
        return carry

    lax.fori_loop(0, nh, body, 0)

    plsc.subcore_barrier()
    pltpu.sync_copy(acc.at[pl.ds(s * _RPS, _RPS)],
                    out.at[c, pl.ds(s * _RPS, _RPS)])


@functools.lru_cache(maxsize=None)
def _get_edge_pass():
    mesh = plsc.VectorSubcoreMesh(core_axis_name="c", subcore_axis_name="s",
                                  num_cores=_NC, num_subcores=_NS)
    return pl.kernel(
        _edge_pass_body,
        out_type=jax.ShapeDtypeStruct((_NC, _NP, _W), jnp.float32),
        mesh=mesh,
        scratch_types=[
            pltpu.VMEM((_CPW, _CHUNK), jnp.int32),
            pltpu.VMEM((_CPW, _CHUNK), jnp.int32),
            pltpu.VMEM((_CHUNK, _W), jnp.float32),
            pltpu.VMEM((_CHUNK, _W), jnp.float32),
            pltpu.VMEM_SHARED((_NP, _W), jnp.float32),
            pltpu.SemaphoreType.DMA,
            pltpu.SemaphoreType.DMA,
        ],
        compiler_params=pltpu.CompilerParams(use_tc_tiling_on_sc=False),
    )


def _embed_body(x_ref, w_ref, b_ref, o_ref):
    h = jnp.dot(x_ref[...], w_ref[...],
                preferred_element_type=jnp.float32) + b_ref[...]
    ones = jnp.ones((_BR, 1), jnp.float32)
    zer = jnp.zeros((_BR, _W - _H - 1), jnp.float32)
    o_ref[...] = jnp.concatenate([h, ones, zer], axis=1)


_embed = pl.pallas_call(
    _embed_body,
    grid=(_NP // _BR,),
    in_specs=[
        pl.BlockSpec((_BR, _D), lambda i: (i, 0)),
        pl.BlockSpec((_D, _H), lambda i: (0, 0)),
        pl.BlockSpec((1, _H), lambda i: (0, 0)),
    ],
    out_specs=pl.BlockSpec((_BR, _W), lambda i: (i, 0)),
    out_shape=jax.ShapeDtypeStruct((_NP, _W), jnp.float32),
)


def _new_h(he_ref, p_ref, wma_ref, wmb_ref, bm_ref, wua_ref, wub_ref, bu_ref,
           g_ref, be_ref):
    h = he_ref[...][:, :_H]
    p0 = p_ref[0]
    p1 = p_ref[1]
    G = p0[:, :_H] + p1[:, :_H]
    cnt = p0[:, _H:_H + 1] + p1[:, _H:_H + 1]
    cntc = jnp.maximum(cnt, 1.0)
    aggr = (jnp.dot(G, wma_ref[...], preferred_element_type=jnp.float32)
            + cnt * jnp.dot(h, wmb_ref[...], preferred_element_type=jnp.float32)
            + cnt * bm_ref[...]) / cntc
    upd = (jnp.dot(h, wua_ref[...], preferred_element_type=jnp.float32)
           + jnp.dot(aggr, wub_ref[...], preferred_element_type=jnp.float32)
           + bu_ref[...])
    y = h + upd
    mu = jnp.mean(y, axis=1, keepdims=True)
    var = jnp.mean((y - mu) ** 2, axis=1, keepdims=True)
    return (y - mu) * lax.rsqrt(var + 1e-5) * g_ref[...] + be_ref[...]


def _layer_body(he_ref, p_ref, wma_ref, wmb_ref, bm_ref, wua_ref, wub_ref,
                bu_ref, g_ref, be_ref, o_ref):
    hn = _new_h(he_ref, p_ref, wma_ref, wmb_ref, bm_ref, wua_ref, wub_ref,
                bu_ref, g_ref, be_ref)
    ones = jnp.ones((_BR, 1), jnp.float32)
    zer = jnp.zeros((_BR, _W - _H - 1), jnp.float32)
    o_ref[...] = jnp.concatenate([hn, ones, zer], axis=1)


def _final_body(he_ref, p_ref, wma_ref, wmb_ref, bm_ref, wua_ref, wub_ref,
                bu_ref, g_ref, be_ref, wo1_ref, bo1_ref, wo2_ref, bo2_ref,
                o_ref):
    hn = _new_h(he_ref, p_ref, wma_ref, wmb_ref, bm_ref, wua_ref, wub_ref,
                bu_ref, g_ref, be_ref)
    z = jnp.dot(hn, wo1_ref[...], preferred_element_type=jnp.float32) \
        + bo1_ref[...]
    z = 0.5 * z * (1.0 + lax.erf(z * 0.7071067811865476))
    z = jnp.dot(z, wo2_ref[...], preferred_element_type=jnp.float32) \
        + bo2_ref[...]
    o_ref[...] = 1.0 / (1.0 + jnp.exp(-z))


_layer_specs = [
    pl.BlockSpec((_BR, _W), lambda i: (i, 0)),
    pl.BlockSpec((_NC, _BR, _W), lambda i: (0, i, 0)),
    pl.BlockSpec((_H, _H), lambda i: (0, 0)),
    pl.BlockSpec((_H, _H), lambda i: (0, 0)),
    pl.BlockSpec((1, _H), lambda i: (0, 0)),
    pl.BlockSpec((_H, _H), lambda i: (0, 0)),
    pl.BlockSpec((_H, _H), lambda i: (0, 0)),
    pl.BlockSpec((1, _H), lambda i: (0, 0)),
    pl.BlockSpec((1, _H), lambda i: (0, 0)),
    pl.BlockSpec((1, _H), lambda i: (0, 0)),
]

_layer = pl.pallas_call(
    _layer_body,
    grid=(_NP // _BR,),
    in_specs=_layer_specs,
    out_specs=pl.BlockSpec((_BR, _W), lambda i: (i, 0)),
    out_shape=jax.ShapeDtypeStruct((_NP, _W), jnp.float32),
)

_final = pl.pallas_call(
    _final_body,
    grid=(_NP // _BR,),
    in_specs=_layer_specs + [
        pl.BlockSpec((_H, _H // 2), lambda i: (0, 0)),
        pl.BlockSpec((1, _H // 2), lambda i: (0, 0)),
        pl.BlockSpec((_H // 2, 1), lambda i: (0, 0)),
        pl.BlockSpec((1, 1), lambda i: (0, 0)),
    ],
    out_specs=pl.BlockSpec((_BR, 1), lambda i: (i, 0)),
    out_shape=jax.ShapeDtypeStruct((_NP, 1), jnp.float32),
)


def kernel(x, edge_index, W_in, b_in, W_msg0, b_msg0, W_upd0, b_upd0, g0, be0,
           W_msg1, b_msg1, W_upd1, b_upd1, g1, be1, W_o1, b_o1, W_o2, b_o2):
    row = edge_index[0]
    col = edge_index[1]
    pad = jnp.full((_EP - _E,), _N, jnp.int32)
    row2d = jnp.concatenate([row, pad]).reshape(_EP // _CHUNK, _CHUNK)
    col2d = jnp.concatenate([col, pad]).reshape(_EP // _CHUNK, _CHUNK)
    xp = jnp.zeros((_NP, _D), jnp.float32).at[:_N].set(x)
    zrows = jnp.zeros((_RPS, _W), jnp.float32)

    edge_pass = _get_edge_pass()
    h0e = _embed(xp, W_in, b_in.reshape(1, _H))
    p0 = edge_pass(h0e, row2d, col2d, zrows)
    h1e = _layer(h0e, p0,
                 W_msg0[:_H], W_msg0[_H:], b_msg0.reshape(1, _H),
                 W_upd0[:_H], W_upd0[_H:], b_upd0.reshape(1, _H),
                 g0.reshape(1, _H), be0.reshape(1, _H))
    p1 = edge_pass(h1e, row2d, col2d, zrows)
    out = _final(h1e, p1,
                 W_msg1[:_H], W_msg1[_H:], b_msg1.reshape(1, _H),
                 W_upd1[:_H], W_upd1[_H:], b_upd1.reshape(1, _H),
                 g1.reshape(1, _H), be1.reshape(1, _H),
                 W_o1, b_o1.reshape(1, _H // 2), W_o2, b_o2.reshape(1, 1))
    return out[:_N, 0]


# speedup vs baseline: 10.6910x; 1.2120x over previous
import functools

import jax
import jax.numpy as jnp
from jax import lax
from jax.experimental import pallas as pl
from jax.experimental.pallas import tpu as pltpu
from jax.experimental.pallas import tpu_sc as plsc

_N = 10000
_E = 640000
_D = 128
_H = 64
_W = 80
_NP = 10240

_NC = 2
_NS = 16
_NW = _NC * _NS
_CHUNK = 128
_CPW0 = 232
_CPW1 = 88
_CPP = _CPW0 + _CPW1
_EP = _NS * _CPP * _CHUNK
_EROWS = 5376
_RPS = _NP // _NS

_BR = 256


def _edge_pass_body(hext, row2d, col2d, zrows, out, idx_r, idx_c, rows0, rows1,
                    acc, gsem0, gsem1):
    c = lax.axis_index("c")
    s = lax.axis_index("s")
    base = s * _CPP + c * _CPW0
    nh = jnp.where(c == 0, _CPW0 // 2, _CPW1 // 2)

    pltpu.sync_copy(zrows, acc.at[pl.ds(s * _RPS, _RPS)])
    pltpu.sync_copy(row2d.at[pl.ds(base, _CPW0)], idx_r)
    pltpu.sync_copy(col2d.at[pl.ds(base, _CPW0)], idx_c)
    plsc.subcore_barrier()

    pltpu.async_copy(hext.at[idx_r.at[0]], rows0, gsem0)

    def body(i, carry):
        j = 2 * i
        pltpu.make_async_copy(hext.at[idx_r.at[j]], rows0, gsem0).wait()
        pltpu.async_copy(hext.at[idx_r.at[j + 1]], rows1, gsem1)
        pltpu.sync_copy(rows0, acc.at[idx_c.at[j]], add=True)
        pltpu.make_async_copy(hext.at[idx_r.at[j + 1]], rows1, gsem1).wait()

        @pl.when(i < nh - 1)
        def _():
            pltpu.async_copy(hext.at[idx_r.at[j + 2]], rows0, gsem0)

        pltpu.sync_copy(rows1, acc.at[idx_c.at[j + 1]], add=True)
        return carry

    lax.fori_loop(0, nh, body, 0, unroll=False)

    plsc.subcore_barrier()
    pltpu.sync_copy(acc.at[pl.ds(s * _RPS, _RPS)],
                    out.at[c, pl.ds(s * _RPS, _RPS)])


@functools.lru_cache(maxsize=None)
def _get_edge_pass():
    mesh = plsc.VectorSubcoreMesh(core_axis_name="c", subcore_axis_name="s",
                                  num_cores=_NC, num_subcores=_NS)
    return pl.kernel(
        _edge_pass_body,
        out_type=jax.ShapeDtypeStruct((_NC, _NP, _W), jnp.float32),
        mesh=mesh,
        scratch_types=[
            pltpu.VMEM((_CPW0, _CHUNK), jnp.int32),
            pltpu.VMEM((_CPW0, _CHUNK), jnp.int32),
            pltpu.VMEM((_CHUNK, _W), jnp.float32),
            pltpu.VMEM((_CHUNK, _W), jnp.float32),
            pltpu.VMEM_SHARED((_NP, _W), jnp.float32),
            pltpu.SemaphoreType.DMA,
            pltpu.SemaphoreType.DMA,
        ],
        compiler_params=pltpu.CompilerParams(use_tc_tiling_on_sc=False),
    )


def _embed_body(x_ref, w_ref, b_ref, o_ref):
    h = jnp.dot(x_ref[...], w_ref[...],
                preferred_element_type=jnp.float32) + b_ref[...]
    ones = jnp.ones((_BR, 1), jnp.float32)
    zer = jnp.zeros((_BR, _W - _H - 1), jnp.float32)
    o_ref[...] = jnp.concatenate([h, ones, zer], axis=1)


_embed = pl.pallas_call(
    _embed_body,
    grid=(_NP // _BR,),
    in_specs=[
        pl.BlockSpec((_BR, _D), lambda i: (i, 0)),
        pl.BlockSpec((_D, _H), lambda i: (0, 0)),
        pl.BlockSpec((1, _H), lambda i: (0, 0)),
    ],
    out_specs=pl.BlockSpec((_BR, _W), lambda i: (i, 0)),
    out_shape=jax.ShapeDtypeStruct((_NP, _W), jnp.float32),
)


def _new_h(he_ref, p_ref, wma_ref, wmb_ref, bm_ref, wua_ref, wub_ref, bu_ref,
           g_ref, be_ref):
    h = he_ref[...][:, :_H]
    p0 = p_ref[0]
    p1 = p_ref[1]
    G = p0[:, :_H] + p1[:, :_H]
    cnt = p0[:, _H:_H + 1] + p1[:, _H:_H + 1]
    cntc = jnp.maximum(cnt, 1.0)
    aggr = (jnp.dot(G, wma_ref[...], preferred_element_type=jnp.float32)
            + cnt * jnp.dot(h, wmb_ref[...], preferred_element_type=jnp.float32)
            + cnt * bm_ref[...]) / cntc
    upd = (jnp.dot(h, wua_ref[...], preferred_element_type=jnp.float32)
           + jnp.dot(aggr, wub_ref[...], preferred_element_type=jnp.float32)
           + bu_ref[...])
    y = h + upd
    mu = jnp.mean(y, axis=1, keepdims=True)
    var = jnp.mean((y - mu) ** 2, axis=1, keepdims=True)
    return (y - mu) * lax.rsqrt(var + 1e-5) * g_ref[...] + be_ref[...]


def _layer_body(he_ref, p_ref, wma_ref, wmb_ref, bm_ref, wua_ref, wub_ref,
                bu_ref, g_ref, be_ref, o_ref):
    hn = _new_h(he_ref, p_ref, wma_ref, wmb_ref, bm_ref, wua_ref, wub_ref,
                bu_ref, g_ref, be_ref)
    ones = jnp.ones((_BR, 1), jnp.float32)
    zer = jnp.zeros((_BR, _W - _H - 1), jnp.float32)
    o_ref[...] = jnp.concatenate([hn, ones, zer], axis=1)


def _final_body(he_ref, p_ref, wma_ref, wmb_ref, bm_ref, wua_ref, wub_ref,
                bu_ref, g_ref, be_ref, wo1_ref, bo1_ref, wo2_ref, bo2_ref,
                o_ref):
    hn = _new_h(he_ref, p_ref, wma_ref, wmb_ref, bm_ref, wua_ref, wub_ref,
                bu_ref, g_ref, be_ref)
    z = jnp.dot(hn, wo1_ref[...], preferred_element_type=jnp.float32) \
        + bo1_ref[...]
    z = 0.5 * z * (1.0 + lax.erf(z * 0.7071067811865476))
    z = jnp.dot(z, wo2_ref[...], preferred_element_type=jnp.float32) \
        + bo2_ref[...]
    o_ref[...] = 1.0 / (1.0 + jnp.exp(-z))


_layer_specs = [
    pl.BlockSpec((_BR, _W), lambda i: (i, 0)),
    pl.BlockSpec((_NC, _BR, _W), lambda i: (0, i, 0)),
    pl.BlockSpec((_H, _H), lambda i: (0, 0)),
    pl.BlockSpec((_H, _H), lambda i: (0, 0)),
    pl.BlockSpec((1, _H), lambda i: (0, 0)),
    pl.BlockSpec((_H, _H), lambda i: (0, 0)),
    pl.BlockSpec((_H, _H), lambda i: (0, 0)),
    pl.BlockSpec((1, _H), lambda i: (0, 0)),
    pl.BlockSpec((1, _H), lambda i: (0, 0)),
    pl.BlockSpec((1, _H), lambda i: (0, 0)),
]

_layer = pl.pallas_call(
    _layer_body,
    grid=(_NP // _BR,),
    in_specs=_layer_specs,
    out_specs=pl.BlockSpec((_BR, _W), lambda i: (i, 0)),
    out_shape=jax.ShapeDtypeStruct((_NP, _W), jnp.float32),
)

_final = pl.pallas_call(
    _final_body,
    grid=(_NP // _BR,),
    in_specs=_layer_specs + [
        pl.BlockSpec((_H, _H // 2), lambda i: (0, 0)),
        pl.BlockSpec((1, _H // 2), lambda i: (0, 0)),
        pl.BlockSpec((_H // 2, 1), lambda i: (0, 0)),
        pl.BlockSpec((1, 1), lambda i: (0, 0)),
    ],
    out_specs=pl.BlockSpec((_BR, 1), lambda i: (i, 0)),
    out_shape=jax.ShapeDtypeStruct((_NP, 1), jnp.float32),
)


def kernel(x, edge_index, W_in, b_in, W_msg0, b_msg0, W_upd0, b_upd0, g0, be0,
           W_msg1, b_msg1, W_upd1, b_upd1, g1, be1, W_o1, b_o1, W_o2, b_o2):
    row = edge_index[0]
    col = edge_index[1]
    pad = jnp.full((_EROWS * _CHUNK - _E,), _N, jnp.int32)
    row2d = jnp.concatenate([row, pad]).reshape(_EROWS, _CHUNK)
    col2d = jnp.concatenate([col, pad]).reshape(_EROWS, _CHUNK)
    xp = jnp.zeros((_NP, _D), jnp.float32).at[:_N].set(x)
    zrows = jnp.zeros((_RPS, _W), jnp.float32)

    edge_pass = _get_edge_pass()
    h0e = _embed(xp, W_in, b_in.reshape(1, _H))
    p0 = edge_pass(h0e, row2d, col2d, zrows)
    h1e = _layer(h0e, p0,
                 W_msg0[:_H], W_msg0[_H:], b_msg0.reshape(1, _H),
                 W_upd0[:_H], W_upd0[_H:], b_upd0.reshape(1, _H),
                 g0.reshape(1, _H), be0.reshape(1, _H))
    p1 = edge_pass(h1e, row2d, col2d, zrows)
    out = _final(h1e, p1,
                 W_msg1[:_H], W_msg1[_H:], b_msg1.reshape(1, _H),
                 W_upd1[:_H], W_upd1[_H:], b_upd1.reshape(1, _H),
                 g1.reshape(1, _H), be1.reshape(1, _H),
                 W_o1, b_o1.reshape(1, _H // 2), W_o2, b_o2.reshape(1, 1))
    return out[:_N, 0]


# speedup vs baseline: 11.1767x; 1.0454x over previous
import functools

import jax
import jax.numpy as jnp
from jax import lax
from jax.experimental import pallas as pl
from jax.experimental.pallas import tpu as pltpu
from jax.experimental.pallas import tpu_sc as plsc

_N = 10000
_E = 640000
_D = 128
_H = 64
_W = 80
_NP = 10240

_NC = 2
_NS = 16
_NW = _NC * _NS
_CHUNK = 128
_CPW0 = 240
_CPW1 = 80
_CPP = _CPW0 + _CPW1
_EP = _NS * _CPP * _CHUNK
_EROWS = 5376
_RPS = _NP // _NS

_BR = 256


_NBUF = 4


def _edge_pass_body(hext, row2d, col2d, zrows, out, idx_r, idx_c, r0, r1, r2,
                    r3, acc, g0, g1, g2, g3):
    rows = [r0, r1, r2, r3]
    gsems = [g0, g1, g2, g3]
    c = lax.axis_index("c")
    s = lax.axis_index("s")
    base = s * _CPP + c * _CPW0
    nh = jnp.where(c == 0, _CPW0 // 2, _CPW1 // 2)

    pltpu.sync_copy(zrows, acc.at[pl.ds(s * _RPS, _RPS)])

    for h in range(2):
        pltpu.sync_copy(row2d.at[pl.ds(base + h * nh, _CPW0 // 2)], idx_r)
        pltpu.sync_copy(col2d.at[pl.ds(base + h * nh, _CPW0 // 2)], idx_c)
        if h == 0:
            plsc.subcore_barrier()

        for b in range(_NBUF):
            pltpu.async_copy(hext.at[idx_r.at[b]], rows[b], gsems[b])

        def body(i, carry):
            j = _NBUF * i
            for b in range(_NBUF):
                pltpu.make_async_copy(hext.at[idx_r.at[j + b]], rows[b],
                                      gsems[b]).wait()
                pltpu.sync_copy(rows[b], acc.at[idx_c.at[j + b]], add=True)

                @pl.when(j + b + _NBUF < nh)
                def _():
                    pltpu.async_copy(hext.at[idx_r.at[j + b + _NBUF]],
                                     rows[b], gsems[b])

            return carry

        lax.fori_loop(0, nh // _NBUF, body, 0, unroll=False)

    plsc.subcore_barrier()
    pltpu.sync_copy(acc.at[pl.ds(s * _RPS, _RPS)],
                    out.at[c, pl.ds(s * _RPS, _RPS)])


@functools.lru_cache(maxsize=None)
def _get_edge_pass():
    mesh = plsc.VectorSubcoreMesh(core_axis_name="c", subcore_axis_name="s",
                                  num_cores=_NC, num_subcores=_NS)
    return pl.kernel(
        _edge_pass_body,
        out_type=jax.ShapeDtypeStruct((_NC, _NP, _W), jnp.float32),
        mesh=mesh,
        scratch_types=[
            pltpu.VMEM((_CPW0 // 2, _CHUNK), jnp.int32),
            pltpu.VMEM((_CPW0 // 2, _CHUNK), jnp.int32),
            pltpu.VMEM((_CHUNK, _W), jnp.float32),
            pltpu.VMEM((_CHUNK, _W), jnp.float32),
            pltpu.VMEM((_CHUNK, _W), jnp.float32),
            pltpu.VMEM((_CHUNK, _W), jnp.float32),
            pltpu.VMEM_SHARED((_NP, _W), jnp.float32),
            pltpu.SemaphoreType.DMA,
            pltpu.SemaphoreType.DMA,
            pltpu.SemaphoreType.DMA,
            pltpu.SemaphoreType.DMA,
        ],
        compiler_params=pltpu.CompilerParams(use_tc_tiling_on_sc=False),
    )


def _embed_body(x_ref, w_ref, b_ref, o_ref):
    h = jnp.dot(x_ref[...], w_ref[...],
                preferred_element_type=jnp.float32) + b_ref[...]
    ones = jnp.ones((_BR, 1), jnp.float32)
    zer = jnp.zeros((_BR, _W - _H - 1), jnp.float32)
    o_ref[...] = jnp.concatenate([h, ones, zer], axis=1)


_embed = pl.pallas_call(
    _embed_body,
    grid=(_NP // _BR,),
    in_specs=[
        pl.BlockSpec((_BR, _D), lambda i: (i, 0)),
        pl.BlockSpec((_D, _H), lambda i: (0, 0)),
        pl.BlockSpec((1, _H), lambda i: (0, 0)),
    ],
    out_specs=pl.BlockSpec((_BR, _W), lambda i: (i, 0)),
    out_shape=jax.ShapeDtypeStruct((_NP, _W), jnp.float32),
)


def _new_h(he_ref, p_ref, wma_ref, wmb_ref, bm_ref, wua_ref, wub_ref, bu_ref,
           g_ref, be_ref):
    h = he_ref[...][:, :_H]
    p0 = p_ref[0]
    p1 = p_ref[1]
    G = p0[:, :_H] + p1[:, :_H]
    cnt = p0[:, _H:_H + 1] + p1[:, _H:_H + 1]
    cntc = jnp.maximum(cnt, 1.0)
    aggr = (jnp.dot(G, wma_ref[...], preferred_element_type=jnp.float32)
            + cnt * jnp.dot(h, wmb_ref[...], preferred_element_type=jnp.float32)
            + cnt * bm_ref[...]) / cntc
    upd = (jnp.dot(h, wua_ref[...], preferred_element_type=jnp.float32)
           + jnp.dot(aggr, wub_ref[...], preferred_element_type=jnp.float32)
           + bu_ref[...])
    y = h + upd
    mu = jnp.mean(y, axis=1, keepdims=True)
    var = jnp.mean((y - mu) ** 2, axis=1, keepdims=True)
    return (y - mu) * lax.rsqrt(var + 1e-5) * g_ref[...] + be_ref[...]


def _layer_body(he_ref, p_ref, wma_ref, wmb_ref, bm_ref, wua_ref, wub_ref,
                bu_ref, g_ref, be_ref, o_ref):
    hn = _new_h(he_ref, p_ref, wma_ref, wmb_ref, bm_ref, wua_ref, wub_ref,
                bu_ref, g_ref, be_ref)
    ones = jnp.ones((_BR, 1), jnp.float32)
    zer = jnp.zeros((_BR, _W - _H - 1), jnp.float32)
    o_ref[...] = jnp.concatenate([hn, ones, zer], axis=1)


def _final_body(he_ref, p_ref, wma_ref, wmb_ref, bm_ref, wua_ref, wub_ref,
                bu_ref, g_ref, be_ref, wo1_ref, bo1_ref, wo2_ref, bo2_ref,
                o_ref):
    hn = _new_h(he_ref, p_ref, wma_ref, wmb_ref, bm_ref, wua_ref, wub_ref,
                bu_ref, g_ref, be_ref)
    z = jnp.dot(hn, wo1_ref[...], preferred_element_type=jnp.float32) \
        + bo1_ref[...]
    z = 0.5 * z * (1.0 + lax.erf(z * 0.7071067811865476))
    z = jnp.dot(z, wo2_ref[...], preferred_element_type=jnp.float32) \
        + bo2_ref[...]
    o_ref[...] = 1.0 / (1.0 + jnp.exp(-z))


_layer_specs = [
    pl.BlockSpec((_BR, _W), lambda i: (i, 0)),
    pl.BlockSpec((_NC, _BR, _W), lambda i: (0, i, 0)),
    pl.BlockSpec((_H, _H), lambda i: (0, 0)),
    pl.BlockSpec((_H, _H), lambda i: (0, 0)),
    pl.BlockSpec((1, _H), lambda i: (0, 0)),
    pl.BlockSpec((_H, _H), lambda i: (0, 0)),
    pl.BlockSpec((_H, _H), lambda i: (0, 0)),
    pl.BlockSpec((1, _H), lambda i: (0, 0)),
    pl.BlockSpec((1, _H), lambda i: (0, 0)),
    pl.BlockSpec((1, _H), lambda i: (0, 0)),
]

_layer = pl.pallas_call(
    _layer_body,
    grid=(_NP // _BR,),
    in_specs=_layer_specs,
    out_specs=pl.BlockSpec((_BR, _W), lambda i: (i, 0)),
    out_shape=jax.ShapeDtypeStruct((_NP, _W), jnp.float32),
)

_final = pl.pallas_call(
    _final_body,
    grid=(_NP // _BR,),
    in_specs=_layer_specs + [
        pl.BlockSpec((_H, _H // 2), lambda i: (0, 0)),
        pl.BlockSpec((1, _H // 2), lambda i: (0, 0)),
        pl.BlockSpec((_H // 2, 1), lambda i: (0, 0)),
        pl.BlockSpec((1, 1), lambda i: (0, 0)),
    ],
    out_specs=pl.BlockSpec((_BR, 1), lambda i: (i, 0)),
    out_shape=jax.ShapeDtypeStruct((_NP, 1), jnp.float32),
)


def kernel(x, edge_index, W_in, b_in, W_msg0, b_msg0, W_upd0, b_upd0, g0, be0,
           W_msg1, b_msg1, W_upd1, b_upd1, g1, be1, W_o1, b_o1, W_o2, b_o2):
    row = edge_index[0]
    col = edge_index[1]
    pad = jnp.full((_EROWS * _CHUNK - _E,), _N, jnp.int32)
    row2d = jnp.concatenate([row, pad]).reshape(_EROWS, _CHUNK)
    col2d = jnp.concatenate([col, pad]).reshape(_EROWS, _CHUNK)
    xp = jnp.zeros((_NP, _D), jnp.float32).at[:_N].set(x)
    zrows = jnp.zeros((_RPS, _W), jnp.float32)

    edge_pass = _get_edge_pass()
    h0e = _embed(xp, W_in, b_in.reshape(1, _H))
    p0 = edge_pass(h0e, row2d, col2d, zrows)
    h1e = _layer(h0e, p0,
                 W_msg0[:_H], W_msg0[_H:], b_msg0.reshape(1, _H),
                 W_upd0[:_H], W_upd0[_H:], b_upd0.reshape(1, _H),
                 g0.reshape(1, _H), be0.reshape(1, _H))
    p1 = edge_pass(h1e, row2d, col2d, zrows)
    out = _final(h1e, p1,
                 W_msg1[:_H], W_msg1[_H:], b_msg1.reshape(1, _H),
                 W_upd1[:_H], W_upd1[_H:], b_upd1.reshape(1, _H),
                 g1.reshape(1, _H), be1.reshape(1, _H),
                 W_o1, b_o1.reshape(1, _H // 2), W_o2, b_o2.reshape(1, 1))
    return out[:_N, 0]


# speedup vs baseline: 11.3430x; 1.0149x over previous
import functools

import jax
import jax.numpy as jnp
from jax import lax
from jax.experimental import pallas as pl
from jax.experimental.pallas import tpu as pltpu
from jax.experimental.pallas import tpu_sc as plsc

_N = 10000
_E = 640000
_D = 128
_H = 64
_W = 80
_NP = 10240

_NC = 2
_NS = 16
_NW = _NC * _NS
_CHUNK = 128
_CPW0 = 240
_CPW1 = 80
_CPP = _CPW0 + _CPW1
_EP = _NS * _CPP * _CHUNK
_EROWS = 5376
_RPS = _NP // _NS

_BR = 256


_NBUF = 4
_LEAD = 2


def _edge_pass_body(hext, row2d, col2d, zrows, out, idx_r, idx_c, r0, r1, r2,
                    r3, acc, g0, g1, g2, g3, s0, s1, s2, s3):
    rows = [r0, r1, r2, r3]
    gsems = [g0, g1, g2, g3]
    ssems = [s0, s1, s2, s3]
    c = lax.axis_index("c")
    s = lax.axis_index("s")
    base = s * _CPP + c * _CPW0
    nh = jnp.where(c == 0, _CPW0 // 2, _CPW1 // 2)

    pltpu.sync_copy(zrows, acc.at[pl.ds(s * _RPS, _RPS)])

    for h in range(2):
        pltpu.sync_copy(row2d.at[pl.ds(base + h * nh, _CPW0 // 2)], idx_r)
        pltpu.sync_copy(col2d.at[pl.ds(base + h * nh, _CPW0 // 2)], idx_c)
        if h == 0:
            plsc.subcore_barrier()

        for b in range(_LEAD):
            pltpu.async_copy(hext.at[idx_r.at[b]], rows[b], gsems[b])

        def body(i, carry):
            j0 = _NBUF * i
            for b in range(_NBUF):
                j = j0 + b
                pltpu.make_async_copy(hext.at[idx_r.at[j]], rows[b],
                                      gsems[b]).wait()
                pltpu.async_copy(rows[b], acc.at[idx_c.at[j]], ssems[b],
                                 add=True)
                ba = (b + _LEAD) % _NBUF

                @pl.when(j + _LEAD - _NBUF >= 0)
                def _():
                    pltpu.make_async_copy(
                        rows[ba], acc.at[idx_c.at[j + _LEAD - _NBUF]],
                        ssems[ba]).wait()

                @pl.when(j + _LEAD < nh)
                def _():
                    pltpu.async_copy(hext.at[idx_r.at[j + _LEAD]], rows[ba],
                                     gsems[ba])

            return carry

        lax.fori_loop(0, nh // _NBUF, body, 0, unroll=False)

        for k in range(_LEAD):
            b = _NBUF - _LEAD + k
            pltpu.make_async_copy(rows[b], acc.at[idx_c.at[nh - _LEAD + k]],
                                  ssems[b]).wait()

    plsc.subcore_barrier()
    pltpu.sync_copy(acc.at[pl.ds(s * _RPS, _RPS)],
                    out.at[c, pl.ds(s * _RPS, _RPS)])


@functools.lru_cache(maxsize=None)
def _get_edge_pass():
    mesh = plsc.VectorSubcoreMesh(core_axis_name="c", subcore_axis_name="s",
                                  num_cores=_NC, num_subcores=_NS)
    return pl.kernel(
        _edge_pass_body,
        out_type=jax.ShapeDtypeStruct((_NC, _NP, _W), jnp.float32),
        mesh=mesh,
        scratch_types=[
            pltpu.VMEM((_CPW0 // 2, _CHUNK), jnp.int32),
            pltpu.VMEM((_CPW0 // 2, _CHUNK), jnp.int32),
            pltpu.VMEM((_CHUNK, _W), jnp.float32),
            pltpu.VMEM((_CHUNK, _W), jnp.float32),
            pltpu.VMEM((_CHUNK, _W), jnp.float32),
            pltpu.VMEM((_CHUNK, _W), jnp.float32),
            pltpu.VMEM_SHARED((_NP, _W), jnp.float32),
            pltpu.SemaphoreType.DMA,
            pltpu.SemaphoreType.DMA,
            pltpu.SemaphoreType.DMA,
            pltpu.SemaphoreType.DMA,
            pltpu.SemaphoreType.DMA,
            pltpu.SemaphoreType.DMA,
            pltpu.SemaphoreType.DMA,
            pltpu.SemaphoreType.DMA,
        ],
        compiler_params=pltpu.CompilerParams(use_tc_tiling_on_sc=False),
    )


def _embed_body(x_ref, w_ref, b_ref, o_ref):
    h = jnp.dot(x_ref[...], w_ref[...],
                preferred_element_type=jnp.float32) + b_ref[...]
    ones = jnp.ones((_BR, 1), jnp.float32)
    zer = jnp.zeros((_BR, _W - _H - 1), jnp.float32)
    o_ref[...] = jnp.concatenate([h, ones, zer], axis=1)


_embed = pl.pallas_call(
    _embed_body,
    grid=(_NP // _BR,),
    in_specs=[
        pl.BlockSpec((_BR, _D), lambda i: (i, 0)),
        pl.BlockSpec((_D, _H), lambda i: (0, 0)),
        pl.BlockSpec((1, _H), lambda i: (0, 0)),
    ],
    out_specs=pl.BlockSpec((_BR, _W), lambda i: (i, 0)),
    out_shape=jax.ShapeDtypeStruct((_NP, _W), jnp.float32),
)


def _new_h(he_ref, p_ref, wma_ref, wmb_ref, bm_ref, wua_ref, wub_ref, bu_ref,
           g_ref, be_ref):
    h = he_ref[...][:, :_H]
    p0 = p_ref[0]
    p1 = p_ref[1]
    G = p0[:, :_H] + p1[:, :_H]
    cnt = p0[:, _H:_H + 1] + p1[:, _H:_H + 1]
    cntc = jnp.maximum(cnt, 1.0)
    aggr = (jnp.dot(G, wma_ref[...], preferred_element_type=jnp.float32)
            + cnt * jnp.dot(h, wmb_ref[...], preferred_element_type=jnp.float32)
            + cnt * bm_ref[...]) / cntc
    upd = (jnp.dot(h, wua_ref[...], preferred_element_type=jnp.float32)
           + jnp.dot(aggr, wub_ref[...], preferred_element_type=jnp.float32)
           + bu_ref[...])
    y = h + upd
    mu = jnp.mean(y, axis=1, keepdims=True)
    var = jnp.mean((y - mu) ** 2, axis=1, keepdims=True)
    return (y - mu) * lax.rsqrt(var + 1e-5) * g_ref[...] + be_ref[...]


def _layer_body(he_ref, p_ref, wma_ref, wmb_ref, bm_ref, wua_ref, wub_ref,
                bu_ref, g_ref, be_ref, o_ref):
    hn = _new_h(he_ref, p_ref, wma_ref, wmb_ref, bm_ref, wua_ref, wub_ref,
                bu_ref, g_ref, be_ref)
    ones = jnp.ones((_BR, 1), jnp.float32)
    zer = jnp.zeros((_BR, _W - _H - 1), jnp.float32)
    o_ref[...] = jnp.concatenate([hn, ones, zer], axis=1)


def _final_body(he_ref, p_ref, wma_ref, wmb_ref, bm_ref, wua_ref, wub_ref,
                bu_ref, g_ref, be_ref, wo1_ref, bo1_ref, wo2_ref, bo2_ref,
                o_ref):
    hn = _new_h(he_ref, p_ref, wma_ref, wmb_ref, bm_ref, wua_ref, wub_ref,
                bu_ref, g_ref, be_ref)
    z = jnp.dot(hn, wo1_ref[...], preferred_element_type=jnp.float32) \
        + bo1_ref[...]
    z = 0.5 * z * (1.0 + lax.erf(z * 0.7071067811865476))
    z = jnp.dot(z, wo2_ref[...], preferred_element_type=jnp.float32) \
        + bo2_ref[...]
    o_ref[...] = 1.0 / (1.0 + jnp.exp(-z))


_layer_specs = [
    pl.BlockSpec((_BR, _W), lambda i: (i, 0)),
    pl.BlockSpec((_NC, _BR, _W), lambda i: (0, i, 0)),
    pl.BlockSpec((_H, _H), lambda i: (0, 0)),
    pl.BlockSpec((_H, _H), lambda i: (0, 0)),
    pl.BlockSpec((1, _H), lambda i: (0, 0)),
    pl.BlockSpec((_H, _H), lambda i: (0, 0)),
    pl.BlockSpec((_H, _H), lambda i: (0, 0)),
    pl.BlockSpec((1, _H), lambda i: (0, 0)),
    pl.BlockSpec((1, _H), lambda i: (0, 0)),
    pl.BlockSpec((1, _H), lambda i: (0, 0)),
]

_layer = pl.pallas_call(
    _layer_body,
    grid=(_NP // _BR,),
    in_specs=_layer_specs,
    out_specs=pl.BlockSpec((_BR, _W), lambda i: (i, 0)),
    out_shape=jax.ShapeDtypeStruct((_NP, _W), jnp.float32),
)

_final = pl.pallas_call(
    _final_body,
    grid=(_NP // _BR,),
    in_specs=_layer_specs + [
        pl.BlockSpec((_H, _H // 2), lambda i: (0, 0)),
        pl.BlockSpec((1, _H // 2), lambda i: (0, 0)),
        pl.BlockSpec((_H // 2, 1), lambda i: (0, 0)),
        pl.BlockSpec((1, 1), lambda i: (0, 0)),
    ],
    out_specs=pl.BlockSpec((_BR, 1), lambda i: (i, 0)),
    out_shape=jax.ShapeDtypeStruct((_NP, 1), jnp.float32),
)


def kernel(x, edge_index, W_in, b_in, W_msg0, b_msg0, W_upd0, b_upd0, g0, be0,
           W_msg1, b_msg1, W_upd1, b_upd1, g1, be1, W_o1, b_o1, W_o2, b_o2):
    row = edge_index[0]
    col = edge_index[1]
    pad = jnp.full((_EROWS * _CHUNK - _E,), _N, jnp.int32)
    row2d = jnp.concatenate([row, pad]).reshape(_EROWS, _CHUNK)
    col2d = jnp.concatenate([col, pad]).reshape(_EROWS, _CHUNK)
    xp = jnp.zeros((_NP, _D), jnp.float32).at[:_N].set(x)
    zrows = jnp.zeros((_RPS, _W), jnp.float32)

    edge_pass = _get_edge_pass()
    h0e = _embed(xp, W_in, b_in.reshape(1, _H))
    p0 = edge_pass(h0e, row2d, col2d, zrows)
    h1e = _layer(h0e, p0,
                 W_msg0[:_H], W_msg0[_H:], b_msg0.reshape(1, _H),
                 W_upd0[:_H], W_upd0[_H:], b_upd0.reshape(1, _H),
                 g0.reshape(1, _H), be0.reshape(1, _H))
    p1 = edge_pass(h1e, row2d, col2d, zrows)
    out = _final(h1e, p1,
                 W_msg1[:_H], W_msg1[_H:], b_msg1.reshape(1, _H),
                 W_upd1[:_H], W_upd1[_H:], b_upd1.reshape(1, _H),
                 g1.reshape(1, _H), be1.reshape(1, _H),
                 W_o1, b_o1.reshape(1, _H // 2), W_o2, b_o2.reshape(1, 1))
    return out[:_N, 0]


# speedup vs baseline: 14.2605x; 1.2572x over previous
import functools

import jax
import jax.numpy as jnp
from jax import lax
from jax.experimental import pallas as pl
from jax.experimental.pallas import tpu as pltpu
from jax.experimental.pallas import tpu_sc as plsc

_N = 10000
_E = 640000
_D = 128
_H = 64
_W = 80
_NP = 10240

_NC = 2
_NS = 16
_NW = _NC * _NS
_CHUNK = 128
_CPW0 = 240
_CPW1 = 80
_CPP = _CPW0 + _CPW1
_EP = _NS * _CPP * _CHUNK
_EROWS = 5376
_RPS = _NP // _NS

_BR = 256


_NBUF = 4
_LEAD = 2


def _make_edge_pass_body(with_count):
    def _edge_pass_body(htab, row2d, col2d, zrows, zcnt, *rest):
        if with_count:
            (out, out_cnt, idx_r, idx_c, r0, r1, r2, r3, acc, cnt,
             g0, g1, g2, g3, s0, s1, s2, s3) = rest
        else:
            (out, idx_r, idx_c, r0, r1, r2, r3, acc,
             g0, g1, g2, g3, s0, s1, s2, s3) = rest
        rows = [r0, r1, r2, r3]
        gsems = [g0, g1, g2, g3]
        ssems = [s0, s1, s2, s3]
        c = lax.axis_index("c")
        s = lax.axis_index("s")
        base = s * _CPP + c * _CPW0
        nh = jnp.where(c == 0, _CPW0 // 2, _CPW1 // 2)
        ones16 = jnp.full((16,), 1.0, jnp.float32)

        pltpu.sync_copy(zrows, acc.at[pl.ds(s * _RPS, _RPS)])
        if with_count:
            pltpu.sync_copy(zcnt, cnt)

        for h in range(2):
            pltpu.sync_copy(row2d.at[pl.ds(base + h * nh, _CPW0 // 2)], idx_r)
            pltpu.sync_copy(col2d.at[pl.ds(base + h * nh, _CPW0 // 2)], idx_c)
            if h == 0:
                plsc.subcore_barrier()

            for b in range(_LEAD):
                pltpu.async_copy(htab.at[idx_r.at[b]], rows[b], gsems[b])

            def body(i, carry):
                j0 = _NBUF * i
                for b in range(_NBUF):
                    j = j0 + b
                    pltpu.make_async_copy(htab.at[idx_r.at[j]], rows[b],
                                          gsems[b]).wait()
                    pltpu.async_copy(rows[b], acc.at[idx_c.at[j]], ssems[b],
                                     add=True)
                    if with_count:
                        for k in range(_CHUNK // 16):
                            v = idx_c[j, pl.ds(k * 16, 16)]
                            plsc.addupdate_scatter(cnt, [v], ones16)
                    ba = (b + _LEAD) % _NBUF

                    @pl.when(j + _LEAD - _NBUF >= 0)
                    def _():
                        pltpu.make_async_copy(
                            rows[ba], acc.at[idx_c.at[j + _LEAD - _NBUF]],
                            ssems[ba]).wait()

                    @pl.when(j + _LEAD < nh)
                    def _():
                        pltpu.async_copy(htab.at[idx_r.at[j + _LEAD]],
                                         rows[ba], gsems[ba])

                return carry

            lax.fori_loop(0, nh // _NBUF, body, 0, unroll=False)

            for k in range(_LEAD):
                b = _NBUF - _LEAD + k
                pltpu.make_async_copy(rows[b],
                                      acc.at[idx_c.at[nh - _LEAD + k]],
                                      ssems[b]).wait()

        plsc.subcore_barrier()
        pltpu.sync_copy(acc.at[pl.ds(s * _RPS, _RPS)],
                        out.at[c, pl.ds(s * _RPS, _RPS)])
        if with_count:
            pltpu.sync_copy(cnt, out_cnt.at[c, s])

    return _edge_pass_body


@functools.lru_cache(maxsize=None)
def _get_edge_pass(with_count):
    mesh = plsc.VectorSubcoreMesh(core_axis_name="c", subcore_axis_name="s",
                                  num_cores=_NC, num_subcores=_NS)
    out_type = [jax.ShapeDtypeStruct((_NC, _NP, _H), jnp.float32)]
    scratch = [
        pltpu.VMEM((_CPW0 // 2, _CHUNK), jnp.int32),
        pltpu.VMEM((_CPW0 // 2, _CHUNK), jnp.int32),
        pltpu.VMEM((_CHUNK, _H), jnp.float32),
        pltpu.VMEM((_CHUNK, _H), jnp.float32),
        pltpu.VMEM((_CHUNK, _H), jnp.float32),
        pltpu.VMEM((_CHUNK, _H), jnp.float32),
        pltpu.VMEM_SHARED((_NP, _H), jnp.float32),
    ]
    if with_count:
        out_type.append(jax.ShapeDtypeStruct((_NC, _NS, _NP), jnp.float32))
        scratch.append(pltpu.VMEM((_NP,), jnp.float32))
    scratch += [pltpu.SemaphoreType.DMA] * 8
    return pl.kernel(
        _make_edge_pass_body(with_count),
        out_type=tuple(out_type),
        mesh=mesh,
        scratch_types=scratch,
        compiler_params=pltpu.CompilerParams(use_tc_tiling_on_sc=False,
                                             needs_layout_passes=False),
    )


def _embed_body(x_ref, w_ref, b_ref, o_ref):
    o_ref[...] = jnp.dot(x_ref[...], w_ref[...],
                         preferred_element_type=jnp.float32) + b_ref[...]


_embed = pl.pallas_call(
    _embed_body,
    grid=(_NP // _BR,),
    in_specs=[
        pl.BlockSpec((_BR, _D), lambda i: (i, 0)),
        pl.BlockSpec((_D, _H), lambda i: (0, 0)),
        pl.BlockSpec((1, _H), lambda i: (0, 0)),
    ],
    out_specs=pl.BlockSpec((_BR, _H), lambda i: (i, 0)),
    out_shape=jax.ShapeDtypeStruct((_NP, _H), jnp.float32),
)


def _new_h(he_ref, p_ref, pc_ref, wma_ref, wmb_ref, bm_ref, wua_ref, wub_ref,
           bu_ref, g_ref, be_ref):
    h = he_ref[...]
    G = p_ref[0] + p_ref[1]
    cnt = jnp.sum(pc_ref[...], axis=(0, 1))[:, None]
    cntc = jnp.maximum(cnt, 1.0)
    aggr = (jnp.dot(G, wma_ref[...], preferred_element_type=jnp.float32)
            + cnt * jnp.dot(h, wmb_ref[...], preferred_element_type=jnp.float32)
            + cnt * bm_ref[...]) / cntc
    upd = (jnp.dot(h, wua_ref[...], preferred_element_type=jnp.float32)
           + jnp.dot(aggr, wub_ref[...], preferred_element_type=jnp.float32)
           + bu_ref[...])
    y = h + upd
    mu = jnp.mean(y, axis=1, keepdims=True)
    var = jnp.mean((y - mu) ** 2, axis=1, keepdims=True)
    return (y - mu) * lax.rsqrt(var + 1e-5) * g_ref[...] + be_ref[...]


def _layer_body(he_ref, p_ref, pc_ref, wma_ref, wmb_ref, bm_ref, wua_ref,
                wub_ref, bu_ref, g_ref, be_ref, o_ref):
    o_ref[...] = _new_h(he_ref, p_ref, pc_ref, wma_ref, wmb_ref, bm_ref,
                        wua_ref, wub_ref, bu_ref, g_ref, be_ref)


def _final_body(he_ref, p_ref, pc_ref, wma_ref, wmb_ref, bm_ref, wua_ref,
                wub_ref, bu_ref, g_ref, be_ref, wo1_ref, bo1_ref, wo2_ref,
                bo2_ref, o_ref):
    hn = _new_h(he_ref, p_ref, pc_ref, wma_ref, wmb_ref, bm_ref, wua_ref,
                wub_ref, bu_ref, g_ref, be_ref)
    z = jnp.dot(hn, wo1_ref[...], preferred_element_type=jnp.float32) \
        + bo1_ref[...]
    z = 0.5 * z * (1.0 + lax.erf(z * 0.7071067811865476))
    z = jnp.dot(z, wo2_ref[...], preferred_element_type=jnp.float32) \
        + bo2_ref[...]
    o_ref[...] = 1.0 / (1.0 + jnp.exp(-z))


_layer_specs = [
    pl.BlockSpec((_BR, _H), lambda i: (i, 0)),
    pl.BlockSpec((_NC, _BR, _H), lambda i: (0, i, 0)),
    pl.BlockSpec((_NC, _NS, _BR), lambda i: (0, 0, i)),
    pl.BlockSpec((_H, _H), lambda i: (0, 0)),
    pl.BlockSpec((_H, _H), lambda i: (0, 0)),
    pl.BlockSpec((1, _H), lambda i: (0, 0)),
    pl.BlockSpec((_H, _H), lambda i: (0, 0)),
    pl.BlockSpec((_H, _H), lambda i: (0, 0)),
    pl.BlockSpec((1, _H), lambda i: (0, 0)),
    pl.BlockSpec((1, _H), lambda i: (0, 0)),
    pl.BlockSpec((1, _H), lambda i: (0, 0)),
]

_layer = pl.pallas_call(
    _layer_body,
    grid=(_NP // _BR,),
    in_specs=_layer_specs,
    out_specs=pl.BlockSpec((_BR, _H), lambda i: (i, 0)),
    out_shape=jax.ShapeDtypeStruct((_NP, _H), jnp.float32),
)

_final = pl.pallas_call(
    _final_body,
    grid=(_NP // _BR,),
    in_specs=_layer_specs + [
        pl.BlockSpec((_H, _H // 2), lambda i: (0, 0)),
        pl.BlockSpec((1, _H // 2), lambda i: (0, 0)),
        pl.BlockSpec((_H // 2, 1), lambda i: (0, 0)),
        pl.BlockSpec((1, 1), lambda i: (0, 0)),
    ],
    out_specs=pl.BlockSpec((_BR, 1), lambda i: (i, 0)),
    out_shape=jax.ShapeDtypeStruct((_NP, 1), jnp.float32),
)


def kernel(x, edge_index, W_in, b_in, W_msg0, b_msg0, W_upd0, b_upd0, g0, be0,
           W_msg1, b_msg1, W_upd1, b_upd1, g1, be1, W_o1, b_o1, W_o2, b_o2):
    row = edge_index[0]
    col = edge_index[1]
    pad = jnp.full((_EROWS * _CHUNK - _E,), _N, jnp.int32)
    row2d = jnp.concatenate([row, pad]).reshape(_EROWS, _CHUNK)
    col2d = jnp.concatenate([col, pad]).reshape(_EROWS, _CHUNK)
    xp = jnp.zeros((_NP, _D), jnp.float32).at[:_N].set(x)
    zrows = jnp.zeros((_RPS, _H), jnp.float32)
    zcnt = jnp.zeros((_NP,), jnp.float32)

    h0e = _embed(xp, W_in, b_in.reshape(1, _H))
    p0, pc = _get_edge_pass(True)(h0e, row2d, col2d, zrows, zcnt)
    h1e = _layer(h0e, p0, pc,
                 W_msg0[:_H], W_msg0[_H:], b_msg0.reshape(1, _H),
                 W_upd0[:_H], W_upd0[_H:], b_upd0.reshape(1, _H),
                 g0.reshape(1, _H), be0.reshape(1, _H))
    (p1,) = _get_edge_pass(False)(h1e, row2d, col2d, zrows, zcnt)
    out = _final(h1e, p1, pc,
                 W_msg1[:_H], W_msg1[_H:], b_msg1.reshape(1, _H),
                 W_upd1[:_H], W_upd1[_H:], b_upd1.reshape(1, _H),
                 g1.reshape(1, _H), be1.reshape(1, _H),
                 W_o1, b_o1.reshape(1, _H // 2), W_o2, b_o2.reshape(1, 1))
    return out[:_N, 0]


# speedup vs baseline: 14.8844x; 1.0437x over previous
import functools

import jax
import jax.numpy as jnp
from jax import lax
from jax.experimental import pallas as pl
from jax.experimental.pallas import tpu as pltpu
from jax.experimental.pallas import tpu_sc as plsc

_N = 10000
_E = 640000
_D = 128
_H = 64
_W = 80
_NP = 10240

_NC = 2
_NS = 16
_NW = _NC * _NS
_CHUNK = 128
_CPW0 = 240
_CPW1 = 80
_CPP = _CPW0 + _CPW1
_EP = _NS * _CPP * _CHUNK
_EROWS = 5376
_RPS = _NP // _NS

_BR = 512


_NBUF = 4
_LEAD = 2


def _make_edge_pass_body(with_count):
    def _edge_pass_body(htab, row2d, col2d, zrows, zcnt, *rest):
        if with_count:
            (out, out_cnt, idx_r, idx_c, r0, r1, r2, r3, acc, cnt,
             g0, g1, g2, g3, s0, s1, s2, s3) = rest
        else:
            (out, idx_r, idx_c, r0, r1, r2, r3, acc,
             g0, g1, g2, g3, s0, s1, s2, s3) = rest
        rows = [r0, r1, r2, r3]
        gsems = [g0, g1, g2, g3]
        ssems = [s0, s1, s2, s3]
        c = lax.axis_index("c")
        s = lax.axis_index("s")
        base = s * _CPP + c * _CPW0
        nh = jnp.where(c == 0, _CPW0 // 2, _CPW1 // 2)
        ones16 = jnp.full((16,), 1.0, jnp.float32)

        pltpu.sync_copy(zrows, acc.at[pl.ds(s * _RPS, _RPS)])
        if with_count:
            pltpu.sync_copy(zcnt, cnt)

        for h in range(2):
            pltpu.sync_copy(row2d.at[pl.ds(base + h * nh, _CPW0 // 2)], idx_r)
            pltpu.sync_copy(col2d.at[pl.ds(base + h * nh, _CPW0 // 2)], idx_c)
            if h == 0:
                plsc.subcore_barrier()

            for b in range(_LEAD):
                pltpu.async_copy(htab.at[idx_r.at[b]], rows[b], gsems[b])

            def body(i, carry):
                j0 = _NBUF * i
                for b in range(_NBUF):
                    j = j0 + b
                    pltpu.make_async_copy(htab.at[idx_r.at[j]], rows[b],
                                          gsems[b]).wait()
                    pltpu.async_copy(rows[b], acc.at[idx_c.at[j]], ssems[b],
                                     add=True)
                    if with_count:
                        for k in range(_CHUNK // 16):
                            v = idx_c[j, pl.ds(k * 16, 16)]
                            plsc.addupdate_scatter(cnt, [v], ones16)
                    ba = (b + _LEAD) % _NBUF

                    @pl.when(j + _LEAD - _NBUF >= 0)
                    def _():
                        pltpu.make_async_copy(
                            rows[ba], acc.at[idx_c.at[j + _LEAD - _NBUF]],
                            ssems[ba]).wait()

                    @pl.when(j + _LEAD < nh)
                    def _():
                        pltpu.async_copy(htab.at[idx_r.at[j + _LEAD]],
                                         rows[ba], gsems[ba])

                return carry

            lax.fori_loop(0, nh // _NBUF, body, 0, unroll=False)

            for k in range(_LEAD):
                b = _NBUF - _LEAD + k
                pltpu.make_async_copy(rows[b],
                                      acc.at[idx_c.at[nh - _LEAD + k]],
                                      ssems[b]).wait()

        plsc.subcore_barrier()
        pltpu.sync_copy(acc.at[pl.ds(s * _RPS, _RPS)],
                        out.at[c, pl.ds(s * _RPS, _RPS)])
        if with_count:
            pltpu.sync_copy(cnt, out_cnt.at[c, s])

    return _edge_pass_body


@functools.lru_cache(maxsize=None)
def _get_edge_pass(with_count):
    mesh = plsc.VectorSubcoreMesh(core_axis_name="c", subcore_axis_name="s",
                                  num_cores=_NC, num_subcores=_NS)
    out_type = [jax.ShapeDtypeStruct((_NC, _NP, _H), jnp.float32)]
    scratch = [
        pltpu.VMEM((_CPW0 // 2, _CHUNK), jnp.int32),
        pltpu.VMEM((_CPW0 // 2, _CHUNK), jnp.int32),
        pltpu.VMEM((_CHUNK, _H), jnp.float32),
        pltpu.VMEM((_CHUNK, _H), jnp.float32),
        pltpu.VMEM((_CHUNK, _H), jnp.float32),
        pltpu.VMEM((_CHUNK, _H), jnp.float32),
        pltpu.VMEM_SHARED((_NP, _H), jnp.float32),
    ]
    if with_count:
        out_type.append(jax.ShapeDtypeStruct((_NC, _NS, _NP), jnp.float32))
        scratch.append(pltpu.VMEM((_NP,), jnp.float32))
    scratch += [pltpu.SemaphoreType.DMA] * 8
    return pl.kernel(
        _make_edge_pass_body(with_count),
        out_type=tuple(out_type),
        mesh=mesh,
        scratch_types=scratch,
        compiler_params=pltpu.CompilerParams(use_tc_tiling_on_sc=False,
                                             needs_layout_passes=False),
    )


def _embed_body(x_ref, w_ref, b_ref, o_ref):
    o_ref[...] = jnp.dot(x_ref[...], w_ref[...],
                         preferred_element_type=jnp.float32) + b_ref[...]


_embed = pl.pallas_call(
    _embed_body,
    grid=(_NP // _BR,),
    in_specs=[
        pl.BlockSpec((_BR, _D), lambda i: (i, 0)),
        pl.BlockSpec((_D, _H), lambda i: (0, 0)),
        pl.BlockSpec((1, _H), lambda i: (0, 0)),
    ],
    out_specs=pl.BlockSpec((_BR, _H), lambda i: (i, 0)),
    out_shape=jax.ShapeDtypeStruct((_NP, _H), jnp.float32),
)


def _new_h(he_ref, p_ref, pc_ref, wma_ref, wmb_ref, bm_ref, wua_ref, wub_ref,
           bu_ref, g_ref, be_ref):
    h = he_ref[...]
    G = p_ref[0] + p_ref[1]
    cnt = jnp.sum(pc_ref[...], axis=(0, 1))[:, None]
    cntc = jnp.maximum(cnt, 1.0)
    aggr = (jnp.dot(G, wma_ref[...], preferred_element_type=jnp.float32)
            + cnt * jnp.dot(h, wmb_ref[...], preferred_element_type=jnp.float32)
            + cnt * bm_ref[...]) / cntc
    upd = (jnp.dot(h, wua_ref[...], preferred_element_type=jnp.float32)
           + jnp.dot(aggr, wub_ref[...], preferred_element_type=jnp.float32)
           + bu_ref[...])
    y = h + upd
    mu = jnp.mean(y, axis=1, keepdims=True)
    var = jnp.mean((y - mu) ** 2, axis=1, keepdims=True)
    return (y - mu) * lax.rsqrt(var + 1e-5) * g_ref[...] + be_ref[...]


def _layer_body(he_ref, p_ref, pc_ref, wma_ref, wmb_ref, bm_ref, wua_ref,
                wub_ref, bu_ref, g_ref, be_ref, o_ref):
    o_ref[...] = _new_h(he_ref, p_ref, pc_ref, wma_ref, wmb_ref, bm_ref,
                        wua_ref, wub_ref, bu_ref, g_ref, be_ref)


def _final_body(he_ref, p_ref, pc_ref, wma_ref, wmb_ref, bm_ref, wua_ref,
                wub_ref, bu_ref, g_ref, be_ref, wo1_ref, bo1_ref, wo2_ref,
                bo2_ref, o_ref):
    hn = _new_h(he_ref, p_ref, pc_ref, wma_ref, wmb_ref, bm_ref, wua_ref,
                wub_ref, bu_ref, g_ref, be_ref)
    z = jnp.dot(hn, wo1_ref[...], preferred_element_type=jnp.float32) \
        + bo1_ref[...]
    z = 0.5 * z * (1.0 + lax.erf(z * 0.7071067811865476))
    z = jnp.dot(z, wo2_ref[...], preferred_element_type=jnp.float32) \
        + bo2_ref[...]
    o_ref[...] = 1.0 / (1.0 + jnp.exp(-z))


_layer_specs = [
    pl.BlockSpec((_BR, _H), lambda i: (i, 0)),
    pl.BlockSpec((_NC, _BR, _H), lambda i: (0, i, 0)),
    pl.BlockSpec((_NC, _NS, _BR), lambda i: (0, 0, i)),
    pl.BlockSpec((_H, _H), lambda i: (0, 0)),
    pl.BlockSpec((_H, _H), lambda i: (0, 0)),
    pl.BlockSpec((1, _H), lambda i: (0, 0)),
    pl.BlockSpec((_H, _H), lambda i: (0, 0)),
    pl.BlockSpec((_H, _H), lambda i: (0, 0)),
    pl.BlockSpec((1, _H), lambda i: (0, 0)),
    pl.BlockSpec((1, _H), lambda i: (0, 0)),
    pl.BlockSpec((1, _H), lambda i: (0, 0)),
]

_layer = pl.pallas_call(
    _layer_body,
    grid=(_NP // _BR,),
    in_specs=_layer_specs,
    out_specs=pl.BlockSpec((_BR, _H), lambda i: (i, 0)),
    out_shape=jax.ShapeDtypeStruct((_NP, _H), jnp.float32),
)

_final = pl.pallas_call(
    _final_body,
    grid=(_NP // _BR,),
    in_specs=_layer_specs + [
        pl.BlockSpec((_H, _H // 2), lambda i: (0, 0)),
        pl.BlockSpec((1, _H // 2), lambda i: (0, 0)),
        pl.BlockSpec((_H // 2, 1), lambda i: (0, 0)),
        pl.BlockSpec((1, 1), lambda i: (0, 0)),
    ],
    out_specs=pl.BlockSpec((_BR, 1), lambda i: (i, 0)),
    out_shape=jax.ShapeDtypeStruct((_NP, 1), jnp.float32),
)


def kernel(x, edge_index, W_in, b_in, W_msg0, b_msg0, W_upd0, b_upd0, g0, be0,
           W_msg1, b_msg1, W_upd1, b_upd1, g1, be1, W_o1, b_o1, W_o2, b_o2):
    row = edge_index[0]
    col = edge_index[1]
    pad = jnp.full((_EROWS * _CHUNK - _E,), _N, jnp.int32)
    row2d = jnp.concatenate([row, pad]).reshape(_EROWS, _CHUNK)
    col2d = jnp.concatenate([col, pad]).reshape(_EROWS, _CHUNK)
    xp = jnp.zeros((_NP, _D), jnp.float32).at[:_N].set(x)
    zrows = jnp.zeros((_RPS, _H), jnp.float32)
    zcnt = jnp.zeros((_NP,), jnp.float32)

    h0e = _embed(xp, W_in, b_in.reshape(1, _H))
    p0, pc = _get_edge_pass(True)(h0e, row2d, col2d, zrows, zcnt)
    h1e = _layer(h0e, p0, pc,
                 W_msg0[:_H], W_msg0[_H:], b_msg0.reshape(1, _H),
                 W_upd0[:_H], W_upd0[_H:], b_upd0.reshape(1, _H),
                 g0.reshape(1, _H), be0.reshape(1, _H))
    (p1,) = _get_edge_pass(False)(h1e, row2d, col2d, zrows, zcnt)
    out = _final(h1e, p1, pc,
                 W_msg1[:_H], W_msg1[_H:], b_msg1.reshape(1, _H),
                 W_upd1[:_H], W_upd1[_H:], b_upd1.reshape(1, _H),
                 g1.reshape(1, _H), be1.reshape(1, _H),
                 W_o1, b_o1.reshape(1, _H // 2), W_o2, b_o2.reshape(1, 1))
    return out[:_N, 0]


# speedup vs baseline: 14.9154x; 1.0021x over previous
import functools

import jax
import jax.numpy as jnp
from jax import lax
from jax.experimental import pallas as pl
from jax.experimental.pallas import tpu as pltpu
from jax.experimental.pallas import tpu_sc as plsc

_N = 10000
_E = 640000
_D = 128
_H = 64
_NP = 10240

_NC = 2
_NS = 16
_CHUNK = 128
_CPW0 = 240
_CPW1 = 80
_CPP = _CPW0 + _CPW1
_EROWS = 5376
_RPS = _NP // _NS

_BR = 512


_NBUF = 4
_LEAD = 2


def _make_edge_pass_body(with_count):
    def _edge_pass_body(htab, row2d, col2d, zrows, zcnt, *rest):
        if with_count:
            (out, out_cnt, idx_r, idx_c, r0, r1, r2, r3, acc, cnt,
             g0, g1, g2, g3, s0, s1, s2, s3) = rest
        else:
            (out, idx_r, idx_c, r0, r1, r2, r3, acc,
             g0, g1, g2, g3, s0, s1, s2, s3) = rest
        rows = [r0, r1, r2, r3]
        gsems = [g0, g1, g2, g3]
        ssems = [s0, s1, s2, s3]
        c = lax.axis_index("c")
        s = lax.axis_index("s")
        base = s * _CPP + c * _CPW0
        nh = jnp.where(c == 0, _CPW0 // 2, _CPW1 // 2)
        ones16 = jnp.full((16,), 1.0, jnp.float32)

        pltpu.sync_copy(zrows, acc.at[pl.ds(s * _RPS, _RPS)])
        if with_count:
            pltpu.sync_copy(zcnt, cnt)

        for h in range(2):
            pltpu.sync_copy(row2d.at[pl.ds(base + h * nh, _CPW0 // 2)], idx_r)
            pltpu.sync_copy(col2d.at[pl.ds(base + h * nh, _CPW0 // 2)], idx_c)
            if h == 0:
                plsc.subcore_barrier()

            for b in range(_LEAD):
                pltpu.async_copy(htab.at[idx_r.at[b]], rows[b], gsems[b])

            def body(i, carry):
                j0 = _NBUF * i
                for b in range(_NBUF):
                    j = j0 + b
                    pltpu.make_async_copy(htab.at[idx_r.at[j]], rows[b],
                                          gsems[b]).wait()
                    pltpu.async_copy(rows[b], acc.at[idx_c.at[j]], ssems[b],
                                     add=True)
                    if with_count:
                        for k in range(_CHUNK // 16):
                            v = idx_c[j, pl.ds(k * 16, 16)]
                            plsc.addupdate_scatter(cnt, [v], ones16)
                    ba = (b + _LEAD) % _NBUF

                    @pl.when(j + _LEAD - _NBUF >= 0)
                    def _():
                        pltpu.make_async_copy(
                            rows[ba], acc.at[idx_c.at[j + _LEAD - _NBUF]],
                            ssems[ba]).wait()

                    @pl.when(j + _LEAD < nh)
                    def _():
                        pltpu.async_copy(htab.at[idx_r.at[j + _LEAD]],
                                         rows[ba], gsems[ba])

                return carry

            lax.fori_loop(0, nh // _NBUF, body, 0, unroll=False)

            for k in range(_LEAD):
                b = _NBUF - _LEAD + k
                pltpu.make_async_copy(rows[b],
                                      acc.at[idx_c.at[nh - _LEAD + k]],
                                      ssems[b]).wait()

        plsc.subcore_barrier()
        pltpu.sync_copy(acc.at[pl.ds(s * _RPS, _RPS)],
                        out.at[c, pl.ds(s * _RPS, _RPS)])
        if with_count:
            pltpu.sync_copy(cnt, out_cnt.at[c, s])

    return _edge_pass_body


@functools.lru_cache(maxsize=None)
def _get_edge_pass(with_count):
    mesh = plsc.VectorSubcoreMesh(core_axis_name="c", subcore_axis_name="s",
                                  num_cores=_NC, num_subcores=_NS)
    out_type = [jax.ShapeDtypeStruct((_NC, _NP, _H), jnp.float32)]
    scratch = [
        pltpu.VMEM((_CPW0 // 2, _CHUNK), jnp.int32),
        pltpu.VMEM((_CPW0 // 2, _CHUNK), jnp.int32),
        pltpu.VMEM((_CHUNK, _H), jnp.float32),
        pltpu.VMEM((_CHUNK, _H), jnp.float32),
        pltpu.VMEM((_CHUNK, _H), jnp.float32),
        pltpu.VMEM((_CHUNK, _H), jnp.float32),
        pltpu.VMEM_SHARED((_NP, _H), jnp.float32),
    ]
    if with_count:
        out_type.append(jax.ShapeDtypeStruct((_NC, _NS, _NP), jnp.float32))
        scratch.append(pltpu.VMEM((_NP,), jnp.float32))
    scratch += [pltpu.SemaphoreType.DMA] * 8
    return pl.kernel(
        _make_edge_pass_body(with_count),
        out_type=tuple(out_type),
        mesh=mesh,
        scratch_types=scratch,
        compiler_params=pltpu.CompilerParams(use_tc_tiling_on_sc=False,
                                             needs_layout_passes=False),
    )


def _embed_body(x_ref, w_ref, b_ref, o_ref):
    o_ref[...] = jnp.dot(x_ref[...], w_ref[...],
                         preferred_element_type=jnp.float32) + b_ref[...]


_embed = pl.pallas_call(
    _embed_body,
    grid=(_NP // _BR,),
    in_specs=[
        pl.BlockSpec((_BR, _D), lambda i: (i, 0)),
        pl.BlockSpec((_D, _H), lambda i: (0, 0)),
        pl.BlockSpec((1, _H), lambda i: (0, 0)),
    ],
    out_specs=pl.BlockSpec((_BR, _H), lambda i: (i, 0)),
    out_shape=jax.ShapeDtypeStruct((_NP, _H), jnp.float32),
)


def _new_h(he_ref, p_ref, pc_ref, wma_ref, wmb_ref, bm_ref, wua_ref, wub_ref,
           bu_ref, g_ref, be_ref):
    h = he_ref[...]
    G = p_ref[0] + p_ref[1]
    cnt = jnp.sum(pc_ref[...], axis=(0, 1))[:, None]
    cntc = jnp.maximum(cnt, 1.0)
    aggr = (jnp.dot(G, wma_ref[...], preferred_element_type=jnp.float32)
            + cnt * jnp.dot(h, wmb_ref[...], preferred_element_type=jnp.float32)
            + cnt * bm_ref[...]) / cntc
    upd = (jnp.dot(h, wua_ref[...], preferred_element_type=jnp.float32)
           + jnp.dot(aggr, wub_ref[...], preferred_element_type=jnp.float32)
           + bu_ref[...])
    y = h + upd
    mu = jnp.mean(y, axis=1, keepdims=True)
    var = jnp.mean((y - mu) ** 2, axis=1, keepdims=True)
    return (y - mu) * lax.rsqrt(var + 1e-5) * g_ref[...] + be_ref[...]


def _layer_body(he_ref, p_ref, pc_ref, wma_ref, wmb_ref, bm_ref, wua_ref,
                wub_ref, bu_ref, g_ref, be_ref, o_ref):
    o_ref[...] = _new_h(he_ref, p_ref, pc_ref, wma_ref, wmb_ref, bm_ref,
                        wua_ref, wub_ref, bu_ref, g_ref, be_ref)


def _final_body(he_ref, p_ref, pc_ref, wma_ref, wmb_ref, bm_ref, wua_ref,
                wub_ref, bu_ref, g_ref, be_ref, wo1_ref, bo1_ref, wo2_ref,
                bo2_ref, o_ref):
    hn = _new_h(he_ref, p_ref, pc_ref, wma_ref, wmb_ref, bm_ref, wua_ref,
                wub_ref, bu_ref, g_ref, be_ref)
    z = jnp.dot(hn, wo1_ref[...], preferred_element_type=jnp.float32) \
        + bo1_ref[...]
    z = 0.5 * z * (1.0 + lax.erf(z * 0.7071067811865476))
    z = jnp.dot(z, wo2_ref[...], preferred_element_type=jnp.float32) \
        + bo2_ref[...]
    o_ref[...] = 1.0 / (1.0 + jnp.exp(-z))


_layer_specs = [
    pl.BlockSpec((_BR, _H), lambda i: (i, 0)),
    pl.BlockSpec((_NC, _BR, _H), lambda i: (0, i, 0)),
    pl.BlockSpec((_NC, _NS, _BR), lambda i: (0, 0, i)),
    pl.BlockSpec((_H, _H), lambda i: (0, 0)),
    pl.BlockSpec((_H, _H), lambda i: (0, 0)),
    pl.BlockSpec((1, _H), lambda i: (0, 0)),
    pl.BlockSpec((_H, _H), lambda i: (0, 0)),
    pl.BlockSpec((_H, _H), lambda i: (0, 0)),
    pl.BlockSpec((1, _H), lambda i: (0, 0)),
    pl.BlockSpec((1, _H), lambda i: (0, 0)),
    pl.BlockSpec((1, _H), lambda i: (0, 0)),
]

_layer = pl.pallas_call(
    _layer_body,
    grid=(_NP // _BR,),
    in_specs=_layer_specs,
    out_specs=pl.BlockSpec((_BR, _H), lambda i: (i, 0)),
    out_shape=jax.ShapeDtypeStruct((_NP, _H), jnp.float32),
)

_final = pl.pallas_call(
    _final_body,
    grid=(_NP // _BR,),
    in_specs=_layer_specs + [
        pl.BlockSpec((_H, _H // 2), lambda i: (0, 0)),
        pl.BlockSpec((1, _H // 2), lambda i: (0, 0)),
        pl.BlockSpec((_H // 2, 1), lambda i: (0, 0)),
        pl.BlockSpec((1, 1), lambda i: (0, 0)),
    ],
    out_specs=pl.BlockSpec((_BR, 1), lambda i: (i, 0)),
    out_shape=jax.ShapeDtypeStruct((_NP, 1), jnp.float32),
)


def kernel(x, edge_index, W_in, b_in, W_msg0, b_msg0, W_upd0, b_upd0, g0, be0,
           W_msg1, b_msg1, W_upd1, b_upd1, g1, be1, W_o1, b_o1, W_o2, b_o2):
    row = edge_index[0]
    col = edge_index[1]
    pad = jnp.full((_EROWS * _CHUNK - _E,), _N, jnp.int32)
    row2d = jnp.concatenate([row, pad]).reshape(_EROWS, _CHUNK)
    col2d = jnp.concatenate([col, pad]).reshape(_EROWS, _CHUNK)
    xp = jnp.zeros((_NP, _D), jnp.float32).at[:_N].set(x)
    zrows = jnp.zeros((_RPS, _H), jnp.float32)
    zcnt = jnp.zeros((_NP,), jnp.float32)

    h0e = _embed(xp, W_in, b_in.reshape(1, _H))
    p0, pc = _get_edge_pass(True)(h0e, row2d, col2d, zrows, zcnt)
    h1e = _layer(h0e, p0, pc,
                 W_msg0[:_H], W_msg0[_H:], b_msg0.reshape(1, _H),
                 W_upd0[:_H], W_upd0[_H:], b_upd0.reshape(1, _H),
                 g0.reshape(1, _H), be0.reshape(1, _H))
    (p1,) = _get_edge_pass(False)(h1e, row2d, col2d, zrows, zcnt)
    out = _final(h1e, p1, pc,
                 W_msg1[:_H], W_msg1[_H:], b_msg1.reshape(1, _H),
                 W_upd1[:_H], W_upd1[_H:], b_upd1.reshape(1, _H),
                 g1.reshape(1, _H), be1.reshape(1, _H),
                 W_o1, b_o1.reshape(1, _H // 2), W_o2, b_o2.reshape(1, 1))
    return out[:_N, 0]
